# baseline (speedup 1.0000x reference)
import functools

import jax
import jax.numpy as jnp
from jax import lax
from jax.experimental import pallas as pl
from jax.experimental.pallas import tpu as pltpu

B, S = 2, 2048
HS = 2048
QLR = 1536
H = 32
D = 128
R = 64
T = S
TOPK = 2048

S_BLK = 256
_PREC = lax.Precision.HIGHEST
_SCALE = D ** -0.5
_BF = jnp.bfloat16


def _rope(x, c, s):
    x1 = x[:, : R // 2]
    x2 = x[:, R // 2 :]
    rot = jnp.concatenate([-x2, x1], axis=1)
    return x * c + rot * s


def _proj(hidden_states, q_resid, cos, sin, Wq_b, Wk, k_norm_gamma,
          k_norm_beta, Wp):
    b, s, _ = hidden_states.shape
    q = q_resid @ Wq_b.T
    q = q.reshape(b, s, H, D)
    c = cos[:, :, None, :]
    sn = sin[:, :, None, :]

    def rot(x):
        x1, x2 = jnp.split(x, 2, axis=-1)
        return jnp.concatenate([-x2, x1], axis=-1)

    q_pe = q[..., :R] * c + rot(q[..., :R]) * sn
    q = jnp.concatenate([q_pe, q[..., R:]], axis=-1)
    k = hidden_states @ Wk.T
    mu = jnp.mean(k, axis=-1, keepdims=True)
    var = jnp.mean((k - mu) ** 2, axis=-1, keepdims=True)
    k = (k - mu) / jnp.sqrt(var + 1e-6) * k_norm_gamma + k_norm_beta
    k_pe = k[..., :R][:, :, None, :] * c + rot(k[..., :R][:, :, None, :]) * sn
    k = jnp.concatenate([k_pe[:, :, 0, :], k[..., R:]], axis=-1)
    w = (hidden_states @ Wp.T) * (H ** -0.5)
    return q.reshape(b, s, H * D), jnp.swapaxes(k, 1, 2), w


def _score_kernel(q_ref, kT_ref, w_ref, out_ref):
    c_unused = None
    w = w_ref[0].astype(_BF).astype(jnp.float32)
    q = q_ref[0]
    acc = jnp.zeros((S_BLK, T), dtype=jnp.float32)
    for h in range(H):
        q_h = q[:, h * D:(h + 1) * D]
        sc = lax.dot_general(q_h, kT_ref[0], (((1,), (0,)), ((), ())),
                             preferred_element_type=jnp.float32)
        r = jnp.maximum(sc * _SCALE, 0.0).astype(_BF).astype(jnp.float32)
        acc = acc + r * w[:, h:h + 1]
    out_ref[0, :, :] = acc


def _index_scores(hidden_states, q_resid, cos, sin, Wq_b, Wk, k_norm_gamma,
                  k_norm_beta, Wp):
    q, kT, w = _proj(hidden_states, q_resid, cos, sin, Wq_b, Wk,
                     k_norm_gamma, k_norm_beta, Wp)
    scores = pl.pallas_call(
        _score_kernel,
        grid=(B, S // S_BLK),
        in_specs=[
            pl.BlockSpec((1, S_BLK, H * D), lambda b, i: (b, i, 0)),
            pl.BlockSpec((1, D, T), lambda b, i: (b, 0, 0)),
            pl.BlockSpec((1, S_BLK, H), lambda b, i: (b, i, 0)),
        ],
        out_specs=pl.BlockSpec((1, S_BLK, T), lambda b, i: (b, i, 0)),
        out_shape=jax.ShapeDtypeStruct((B, S, T), jnp.float32),
    )(q, kT, w)
    return scores


def kernel(hidden_states, q_resid, cos, sin, Wq_b, Wk, k_norm_gamma,
           k_norm_beta, Wp):
    scores = _index_scores(hidden_states, q_resid, cos, sin, Wq_b, Wk,
                           k_norm_gamma, k_norm_beta, Wp)
    _, idx = lax.top_k(scores, TOPK)
    return idx


# speedup vs baseline: 1.0199x; 1.0199x over previous
import functools

import jax
import jax.numpy as jnp
from jax import lax
from jax.experimental import pallas as pl
from jax.experimental.pallas import tpu as pltpu

B, S = 2, 2048
HS = 2048
QLR = 1536
H = 32
D = 128
R = 64
T = S
TOPK = 2048

S_BLK = 64
_SCALE = D ** -0.5
_BF = jnp.bfloat16


def _proj(hidden_states, q_resid, cos, sin, Wq_b, Wk, k_norm_gamma,
          k_norm_beta, Wp):
    b, s, _ = hidden_states.shape
    q = q_resid @ Wq_b.T
    q = q.reshape(b, s, H, D)
    c = cos[:, :, None, :]
    sn = sin[:, :, None, :]

    def rot(x):
        x1, x2 = jnp.split(x, 2, axis=-1)
        return jnp.concatenate([-x2, x1], axis=-1)

    q_pe = q[..., :R] * c + rot(q[..., :R]) * sn
    q = jnp.concatenate([q_pe, q[..., R:]], axis=-1)
    k = hidden_states @ Wk.T
    mu = jnp.mean(k, axis=-1, keepdims=True)
    var = jnp.mean((k - mu) ** 2, axis=-1, keepdims=True)
    k = (k - mu) / jnp.sqrt(var + 1e-6) * k_norm_gamma + k_norm_beta
    k_pe = k[..., :R][:, :, None, :] * c + rot(k[..., :R][:, :, None, :]) * sn
    k = jnp.concatenate([k_pe[:, :, 0, :], k[..., R:]], axis=-1)
    w = (hidden_states @ Wp.T) * (H ** -0.5)
    return q.reshape(b, s * H, D), jnp.swapaxes(k, 1, 2), w


def _score_kernel(q_ref, kT_ref, w_ref, out_ref):
    q = q_ref[0]
    wv = w_ref[0].astype(_BF).astype(jnp.float32)
    sc = lax.dot_general(q, kT_ref[0], (((1,), (0,)), ((), ())),
                         preferred_element_type=jnp.float32)
    r = jnp.maximum(sc * _SCALE, 0.0).astype(_BF).astype(jnp.float32)
    out_ref[0, :, :] = jnp.sum(r.reshape(S_BLK, H, T) * wv[:, :, None], axis=1)


def _index_scores(hidden_states, q_resid, cos, sin, Wq_b, Wk, k_norm_gamma,
                  k_norm_beta, Wp):
    q, kT, w = _proj(hidden_states, q_resid, cos, sin, Wq_b, Wk,
                     k_norm_gamma, k_norm_beta, Wp)
    scores = pl.pallas_call(
        _score_kernel,
        grid=(B, S // S_BLK),
        in_specs=[
            pl.BlockSpec((1, S_BLK * H, D), lambda b, i: (b, i, 0)),
            pl.BlockSpec((1, D, T), lambda b, i: (b, 0, 0)),
            pl.BlockSpec((1, S_BLK, H), lambda b, i: (b, i, 0)),
        ],
        out_specs=pl.BlockSpec((1, S_BLK, T), lambda b, i: (b, i, 0)),
        out_shape=jax.ShapeDtypeStruct((B, S, T), jnp.float32),
    )(q, kT, w)
    return scores


def kernel(hidden_states, q_resid, cos, sin, Wq_b, Wk, k_norm_gamma,
           k_norm_beta, Wp):
    scores = _index_scores(hidden_states, q_resid, cos, sin, Wq_b, Wk,
                           k_norm_gamma, k_norm_beta, Wp)
    _, idx = lax.top_k(scores, TOPK)
    return idx


# speedup vs baseline: 1.0209x; 1.0010x over previous
import functools

import jax
import jax.numpy as jnp
from jax import lax
from jax.experimental import pallas as pl
from jax.experimental.pallas import tpu as pltpu

B, S = 2, 2048
HS = 2048
QLR = 1536
H = 32
D = 128
R = 64
T = S
TOPK = 2048

S_BLK = 64
_SCALE = D ** -0.5
_BF = jnp.bfloat16


def _proj(hidden_states, q_resid, cos, sin, Wq_b, Wk, k_norm_gamma,
          k_norm_beta, Wp):
    b, s, _ = hidden_states.shape
    q = q_resid @ Wq_b.T
    q = q.reshape(b, s, H, D)
    c = cos[:, :, None, :]
    sn = sin[:, :, None, :]

    def rot(x):
        x1, x2 = jnp.split(x, 2, axis=-1)
        return jnp.concatenate([-x2, x1], axis=-1)

    q_pe = q[..., :R] * c + rot(q[..., :R]) * sn
    q = jnp.concatenate([q_pe, q[..., R:]], axis=-1)
    k = hidden_states @ Wk.T
    mu = jnp.mean(k, axis=-1, keepdims=True)
    var = jnp.mean((k - mu) ** 2, axis=-1, keepdims=True)
    k = (k - mu) / jnp.sqrt(var + 1e-6) * k_norm_gamma + k_norm_beta
    k_pe = k[..., :R][:, :, None, :] * c + rot(k[..., :R][:, :, None, :]) * sn
    k = jnp.concatenate([k_pe[:, :, 0, :], k[..., R:]], axis=-1)
    w = (hidden_states @ Wp.T) * (H ** -0.5)
    wt = w.astype(jnp.bfloat16).astype(jnp.float32)
    w4 = wt.reshape(b, s // S_BLK, S_BLK, H)
    eye = jnp.eye(S_BLK, dtype=jnp.float32)
    wmat = (eye[None, None, :, :, None] * w4[:, :, None, :, :]).reshape(
        b, s // S_BLK, S_BLK, S_BLK * H)
    return q.reshape(b, s * H, D), jnp.swapaxes(k, 1, 2), wmat


def _score_kernel(q_ref, kT_ref, wmat_ref, out_ref):
    q = q_ref[0]
    sc = lax.dot_general(q, kT_ref[0], (((1,), (0,)), ((), ())),
                         preferred_element_type=jnp.float32)
    r = jnp.maximum(sc * _SCALE, 0.0).astype(_BF).astype(jnp.float32)
    out_ref[0, :, :] = lax.dot_general(wmat_ref[0, 0], r,
                                       (((1,), (0,)), ((), ())),
                                       preferred_element_type=jnp.float32)


def _index_scores(hidden_states, q_resid, cos, sin, Wq_b, Wk, k_norm_gamma,
                  k_norm_beta, Wp):
    q, kT, w = _proj(hidden_states, q_resid, cos, sin, Wq_b, Wk,
                     k_norm_gamma, k_norm_beta, Wp)
    scores = pl.pallas_call(
        _score_kernel,
        grid=(B, S // S_BLK),
        in_specs=[
            pl.BlockSpec((1, S_BLK * H, D), lambda b, i: (b, i, 0)),
            pl.BlockSpec((1, D, T), lambda b, i: (b, 0, 0)),
            pl.BlockSpec((1, 1, S_BLK, S_BLK * H), lambda b, i: (b, i, 0, 0)),
        ],
        out_specs=pl.BlockSpec((1, S_BLK, T), lambda b, i: (b, i, 0)),
        out_shape=jax.ShapeDtypeStruct((B, S, T), jnp.float32),
    )(q, kT, w)
    return scores


def kernel(hidden_states, q_resid, cos, sin, Wq_b, Wk, k_norm_gamma,
           k_norm_beta, Wp):
    scores = _index_scores(hidden_states, q_resid, cos, sin, Wq_b, Wk,
                           k_norm_gamma, k_norm_beta, Wp)
    _, idx = lax.top_k(scores, TOPK)
    return idx
